# baseline (speedup 1.0000x reference)
import jax
import jax.numpy as jnp
from jax.experimental import pallas as pl


def _add_body(x_ref, p_ref, o_ref):
    o_ref[...] = x_ref[...] + p_ref[...]


def kernel(x, pos_emb):
    B, S, D = x.shape
    LBLK = 1024
    grid = (S // LBLK,)
    return pl.pallas_call(
        _add_body,
        grid=grid,
        in_specs=[
            pl.BlockSpec((B, LBLK, D), lambda i: (0, i, 0)),
            pl.BlockSpec((LBLK, D), lambda i: (i, 0)),
        ],
        out_specs=pl.BlockSpec((B, LBLK, D), lambda i: (0, i, 0)),
        out_shape=jax.ShapeDtypeStruct(x.shape, x.dtype),
    )(x, pos_emb)


# speedup vs baseline: 1.0006x; 1.0006x over previous
import jax
import jax.numpy as jnp
from jax.experimental import pallas as pl


def _add_body(x_ref, p_ref, o_ref):
    o_ref[...] = x_ref[...] + p_ref[...]


def kernel(x, pos_emb):
    B, S, D = x.shape
    LBLK = 2048
    grid = (S // LBLK, B)
    return pl.pallas_call(
        _add_body,
        grid=grid,
        in_specs=[
            pl.BlockSpec((1, LBLK, D), lambda i, b: (b, i, 0)),
            pl.BlockSpec((LBLK, D), lambda i, b: (i, 0)),
        ],
        out_specs=pl.BlockSpec((1, LBLK, D), lambda i, b: (b, i, 0)),
        out_shape=jax.ShapeDtypeStruct(x.shape, x.dtype),
    )(x, pos_emb)
